# baseline (speedup 1.0000x reference)
import functools

import jax
import jax.numpy as jnp
from jax import lax
from jax.experimental import pallas as pl
from jax.experimental.pallas import tpu as pltpu
from jax.experimental.pallas import tpu_sc as plsc

_N = 65535
_D = 64
_NW = 32
_RPW = 2048
_TAIL = 16


@functools.partial(
    pl.kernel,
    mesh=plsc.VectorSubcoreMesh(core_axis_name="c", subcore_axis_name="s"),
    out_type=jax.ShapeDtypeStruct((_N, _D), jnp.float32),
    scratch_types=[pltpu.VMEM((_TAIL, _D), jnp.float32)],
    compiler_params=pltpu.CompilerParams(use_tc_tiling_on_sc=False),
)
def _copy_all(table_hbm, out_hbm, tail_buf):
    wid = lax.axis_index("s") * 2 + lax.axis_index("c")
    base = wid * _RPW

    @pl.when(wid < _NW - 1)
    def _():
        pltpu.sync_copy(table_hbm.at[pl.ds(base, _RPW)],
                        out_hbm.at[pl.ds(base, _RPW)])

    @pl.when(wid == _NW - 1)
    def _():
        pltpu.sync_copy(table_hbm.at[pl.ds(base, _RPW - 8)],
                        out_hbm.at[pl.ds(base, _RPW - 8)])
        idx = jnp.minimum(
            lax.broadcasted_iota(jnp.int32, (_TAIL,), 0) + (_N - 7), _N - 1)
        pltpu.sync_copy(table_hbm.at[idx], tail_buf)
        pltpu.sync_copy(tail_buf, out_hbm.at[idx])


def kernel(table):
    return _copy_all(table)
